# baseline (speedup 1.0000x reference)
import jax
import jax.numpy as jnp
from jax.experimental import pallas as pl
from jax.experimental.pallas import tpu as pltpu

B, F, T = 128, 80, 4096
NFREQ, NTIME = 2, 10


def _mask_body(x_len_ref, fs_ref, fl_ref, ts_ref, tl_ref, x_ref, o_ref):
    b = pl.program_id(0)
    xl = x_len_ref[b]
    ti = jax.lax.broadcasted_iota(jnp.int32, (1, T), 1)
    tkill = jnp.zeros((1, T), jnp.bool_)
    for i in range(NTIME):
        s = ts_ref[i]
        tkill = tkill | ((ti >= s) & (ti < s + tl_ref[i]))
    tkill = tkill & (ti < xl)
    fi = jax.lax.broadcasted_iota(jnp.int32, (F, 1), 0)
    fkill = jnp.zeros((F, 1), jnp.bool_)
    for i in range(NFREQ):
        s = fs_ref[i]
        fkill = fkill | ((fi >= s) & (fi < s + fl_ref[i]))
    mask = fkill | tkill
    o_ref[0] = jnp.where(mask, jnp.float32(0.0), x_ref[0])


def kernel(x, x_len, freq_starts, freq_lengths, time_starts, time_lengths):
    smem = pl.BlockSpec(memory_space=pltpu.SMEM)
    return pl.pallas_call(
        _mask_body,
        grid=(B,),
        in_specs=[
            smem, smem, smem, smem, smem,
            pl.BlockSpec((1, F, T), lambda b: (b, 0, 0)),
        ],
        out_specs=pl.BlockSpec((1, F, T), lambda b: (b, 0, 0)),
        out_shape=jax.ShapeDtypeStruct((B, F, T), jnp.float32),
    )(x_len, freq_starts, freq_lengths, time_starts, time_lengths, x)


# speedup vs baseline: 1.5627x; 1.5627x over previous
import jax
import jax.numpy as jnp
from jax.experimental import pallas as pl
from jax.experimental.pallas import tpu as pltpu

B, F, T = 128, 80, 4096
NFREQ, NTIME = 2, 10


BB = 4


def _mask_body(x_len_ref, fs_ref, fl_ref, ts_ref, tl_ref, x_ref, o_ref):
    g = pl.program_id(0)
    ti = jax.lax.broadcasted_iota(jnp.int32, (1, T), 1)
    tkill = jnp.zeros((1, T), jnp.bool_)
    for i in range(NTIME):
        s = ts_ref[i]
        tkill = tkill | ((ti >= s) & (ti < s + tl_ref[i]))
    fi = jax.lax.broadcasted_iota(jnp.int32, (F, 1), 0)
    fkill = jnp.zeros((F, 1), jnp.bool_)
    for i in range(NFREQ):
        s = fs_ref[i]
        fkill = fkill | ((fi >= s) & (fi < s + fl_ref[i]))
    for j in range(BB):
        xl = x_len_ref[g * BB + j]
        mask = fkill | (tkill & (ti < xl))
        o_ref[j] = jnp.where(mask, jnp.float32(0.0), x_ref[j])


def kernel(x, x_len, freq_starts, freq_lengths, time_starts, time_lengths):
    smem = pl.BlockSpec(memory_space=pltpu.SMEM)
    return pl.pallas_call(
        _mask_body,
        grid=(B // BB,),
        in_specs=[
            smem, smem, smem, smem, smem,
            pl.BlockSpec((BB, F, T), lambda b: (b, 0, 0)),
        ],
        out_specs=pl.BlockSpec((BB, F, T), lambda b: (b, 0, 0)),
        out_shape=jax.ShapeDtypeStruct((B, F, T), jnp.float32),
    )(x_len, freq_starts, freq_lengths, time_starts, time_lengths, x)
